# Initial kernel scaffold; baseline (speedup 1.0000x reference)
import jax
import jax.numpy as jnp
from jax.experimental import pallas as pl


def kernel(h, edge_index, edge_attr, W_fc, W_attn):
    raise NotImplementedError("write your pallas kernel here")


# speedup vs baseline: 10.7436x; 10.7436x over previous
import functools

import jax
import jax.numpy as jnp
from jax import lax
from jax.experimental import pallas as pl
from jax.experimental.pallas import tpu as pltpu
from jax.experimental.pallas import tpu_sc as plsc

NC, NS, LANES = 2, 16, 16
NW = NC * NS
NPAD = 10240
SLICE = NPAD // NS
K = 80


def _tc_dense(h, w_fc, ws_row, wd_row):
    n, d = h.shape
    br = 2000

    def body(h_ref, w_ref, ws_ref, wd_ref, z_ref, as_ref, ad_ref):
        zb = lax.dot_general(h_ref[...], w_ref[...],
                             dimension_numbers=(((1,), (1,)), ((), ())),
                             preferred_element_type=jnp.float32)
        zb = jnp.where(zb >= 0.0, zb, 0.2 * zb)
        z_ref[...] = zb
        as_ref[...] = jnp.sum(zb * ws_ref[...], axis=1, keepdims=True)
        ad_ref[...] = jnp.sum(zb * wd_ref[...], axis=1, keepdims=True)

    return pl.pallas_call(
        body,
        grid=(n // br,),
        in_specs=[
            pl.BlockSpec((br, d), lambda i: (i, 0)),
            pl.BlockSpec((d, d), lambda i: (0, 0)),
            pl.BlockSpec((1, d), lambda i: (0, 0)),
            pl.BlockSpec((1, d), lambda i: (0, 0)),
        ],
        out_specs=[
            pl.BlockSpec((br, d), lambda i: (i, 0)),
            pl.BlockSpec((br, 1), lambda i: (i, 0)),
            pl.BlockSpec((br, 1), lambda i: (i, 0)),
        ],
        out_shape=[
            jax.ShapeDtypeStruct((n, d), jnp.float32),
            jax.ShapeDtypeStruct((n, 1), jnp.float32),
            jax.ShapeDtypeStruct((n, 1), jnp.float32),
        ],
    )(h, w_fc, ws_row, wd_row)


def _tc_ea(edge_attr, we_row):
    e, de = edge_attr.shape
    be = 8000

    def body(a_ref, w_ref, o_ref):
        o_ref[...] = jnp.sum(a_ref[...] * w_ref[...], axis=1, keepdims=True)

    return pl.pallas_call(
        body,
        grid=(e // be,),
        in_specs=[
            pl.BlockSpec((be, de), lambda i: (i, 0)),
            pl.BlockSpec((1, de), lambda i: (0, 0)),
        ],
        out_specs=pl.BlockSpec((be, 1), lambda i: (i, 0)),
        out_shape=jax.ShapeDtypeStruct((e, 1), jnp.float32),
    )(edge_attr, we_row)


def _sc_edge_logits(src, dst, ea, a_src, a_dst):
    e = src.shape[0]
    n = a_src.shape[0]
    ept = e // NW
    nv = ept // LANES
    mesh = plsc.VectorSubcoreMesh(core_axis_name="c", subcore_axis_name="s",
                                  num_cores=NC, num_subcores=NS)

    @functools.partial(
        pl.kernel,
        out_type=[
            jax.ShapeDtypeStruct((e,), jnp.float32),
            jax.ShapeDtypeStruct((NC, NPAD), jnp.float32),
        ],
        mesh=mesh,
        scratch_types=[
            pltpu.VMEM((NPAD,), jnp.float32),
            pltpu.VMEM((NPAD,), jnp.float32),
            pltpu.VMEM((NPAD,), jnp.float32),
            pltpu.VMEM((ept,), jnp.int32),
            pltpu.VMEM((ept,), jnp.int32),
            pltpu.VMEM((ept,), jnp.float32),
            pltpu.VMEM((ept,), jnp.float32),
            pltpu.VMEM((SLICE,), jnp.float32),
            pltpu.VMEM((SLICE,), jnp.float32),
            pltpu.VMEM_SHARED((NS, NPAD), jnp.float32),
        ],
        compiler_params=pltpu.CompilerParams(needs_layout_passes=False),
    )
    def k(src_h, dst_h, ea_h, as_h, ad_h, x_h, dpart_h,
          asrc_v, adst_v, den_v, src_v, dst_v, ea_v, x_v, red_v, tmp_v, shd):
        cid = lax.axis_index("c")
        sid = lax.axis_index("s")
        base = (cid * NS + sid) * ept
        pltpu.sync_copy(src_h.at[pl.ds(base, ept)], src_v)
        pltpu.sync_copy(dst_h.at[pl.ds(base, ept)], dst_v)
        pltpu.sync_copy(ea_h.at[pl.ds(base, ept)], ea_v)
        pltpu.sync_copy(as_h, asrc_v.at[pl.ds(0, n)])
        pltpu.sync_copy(ad_h, adst_v.at[pl.ds(0, n)])

        zeros = jnp.zeros((LANES,), jnp.float32)

        def zero_step(i, _):
            den_v[pl.ds(i * LANES, LANES)] = zeros
            return 0
        lax.fori_loop(0, NPAD // LANES, zero_step, 0)

        def step(i, _):
            si = src_v[pl.ds(i * LANES, LANES)]
            di = dst_v[pl.ds(i * LANES, LANES)]
            t = (plsc.load_gather(asrc_v, [si]) +
                 plsc.load_gather(adst_v, [di]) +
                 ea_v[pl.ds(i * LANES, LANES)])
            t = jnp.where(t >= 0.0, t, 0.2 * t)
            xv = jnp.exp(t)
            x_v[pl.ds(i * LANES, LANES)] = xv
            plsc.addupdate_scatter(den_v, [di], xv)
            return 0
        lax.fori_loop(0, nv, step, 0)

        pltpu.sync_copy(x_v, x_h.at[pl.ds(base, ept)])
        pltpu.sync_copy(den_v, shd.at[sid])
        plsc.subcore_barrier()

        off = sid * SLICE
        pltpu.sync_copy(shd.at[0, pl.ds(off, SLICE)], red_v)
        for j in range(1, NS):
            pltpu.sync_copy(shd.at[j, pl.ds(off, SLICE)], tmp_v)

            def add_step(i, _):
                sl = pl.ds(i * LANES, LANES)
                red_v[sl] = red_v[sl] + tmp_v[sl]
                return 0
            lax.fori_loop(0, SLICE // LANES, add_step, 0)
        pltpu.sync_copy(red_v, dpart_h.at[cid, pl.ds(off, SLICE)])

    return k(src, dst, ea, a_src, a_dst)


def _sc_scatter(ep, dpart, z):
    n, d = z.shape
    e = ep.shape[0] // 3
    ept = e // NW
    nch = ept // K
    zrows = 16
    mesh = plsc.VectorSubcoreMesh(core_axis_name="c", subcore_axis_name="s",
                                  num_cores=NC, num_subcores=NS)

    @functools.partial(
        pl.kernel,
        out_type=jax.ShapeDtypeStruct((NC, n, d), jnp.float32),
        mesh=mesh,
        scratch_types=[
            pltpu.VMEM((NPAD,), jnp.float32),
            pltpu.VMEM((NPAD,), jnp.float32),
            pltpu.VMEM((3 * K,), jnp.int32),
            pltpu.VMEM((K,), jnp.int32),
            pltpu.VMEM((K,), jnp.int32),
            pltpu.VMEM((K,), jnp.float32),
            pltpu.VMEM((K, 128), jnp.float32),
            pltpu.VMEM((zrows, 128), jnp.float32),
            pltpu.SemaphoreType.DMA,
            pltpu.VMEM_SHARED((n, 128), jnp.float32),
        ],
        compiler_params=pltpu.CompilerParams(needs_layout_passes=False),
    )
    def k(ep_h, dp_h, z_h, hp_h,
          den_v, tmp_v, ebuf, srcc, dstc, al, rows, zb, sem, shacc):
        cid = lax.axis_index("c")
        sid = lax.axis_index("s")
        wid = cid * NS + sid

        pltpu.sync_copy(dp_h.at[0], den_v)
        pltpu.sync_copy(dp_h.at[1], tmp_v)

        def den_step(i, _):
            sl = pl.ds(i * LANES, LANES)
            dv = den_v[sl] + tmp_v[sl]
            den_v[sl] = jnp.where(dv == 0.0, 1.0, dv)
            return 0
        lax.fori_loop(0, NPAD // LANES, den_step, 0)

        zeros = jnp.zeros((LANES,), jnp.float32)

        def zb_step(i, _):
            zb[i // 8, pl.ds((i % 8) * LANES, LANES)] = zeros
            return 0
        lax.fori_loop(0, zrows * 8, zb_step, 0)
        r0 = sid * 624
        nq = jnp.where(sid == NS - 1, 40, 39)

        def zero_out(q, _):
            pltpu.sync_copy(zb, shacc.at[pl.ds(r0 + q * zrows, zrows)])
            return 0
        lax.fori_loop(0, nq, zero_out, 0)
        plsc.subcore_barrier()

        def chunk(c, _):
            pltpu.sync_copy(ep_h.at[pl.ds((wid * nch + c) * 3 * K, 3 * K)], ebuf)

            def cp_step(g, _):
                sl = pl.ds(g * LANES, LANES)
                srcc[sl] = ebuf[pl.ds(g * LANES, LANES)]
                di = ebuf[pl.ds(K + g * LANES, LANES)]
                dstc[sl] = di
                xv = plsc.bitcast(ebuf[pl.ds(2 * K + g * LANES, LANES)],
                                  jnp.float32)
                al[sl] = xv / plsc.load_gather(den_v, [di])
                return 0
            lax.fori_loop(0, K // LANES, cp_step, 0)

            pltpu.async_copy(z_h.at[srcc], rows, sem).wait()

            def row_step(r, _):
                av = plsc.load_gather(al, [jnp.full((LANES,), r, jnp.int32)])
                for f in range(8):
                    sl = pl.ds(f * LANES, LANES)
                    rows[r, sl] = rows[r, sl] * av
                return 0
            lax.fori_loop(0, K, row_step, 0)

            pltpu.sync_copy(rows, shacc.at[dstc], add=True)
            return 0
        lax.fori_loop(0, nch, chunk, 0)
        plsc.subcore_barrier()

        def write_out(q, _):
            pltpu.sync_copy(shacc.at[pl.ds(r0 + q * zrows, zrows)], zb)
            pltpu.sync_copy(zb, hp_h.at[cid, pl.ds(r0 + q * zrows, zrows)])
            return 0
        lax.fori_loop(0, nq, write_out, 0)

    return k(ep, dpart, z)


def _tc_sum(hpart):
    _, n, d = hpart.shape
    br = 2000

    def body(a_ref, b_ref, o_ref):
        o_ref[...] = a_ref[0] + b_ref[0]

    return pl.pallas_call(
        body,
        grid=(n // br,),
        in_specs=[
            pl.BlockSpec((1, br, d), lambda i: (0, i, 0)),
            pl.BlockSpec((1, br, d), lambda i: (1, i, 0)),
        ],
        out_specs=pl.BlockSpec((br, d), lambda i: (i, 0)),
        out_shape=jax.ShapeDtypeStruct((n, d), jnp.float32),
    )(hpart, hpart)


def kernel(h, edge_index, edge_attr, W_fc, W_attn):
    n, d_in = h.shape
    e = edge_index.shape[1]
    d_out = W_fc.shape[0]
    d_edge = edge_attr.shape[1]
    src = edge_index[0]
    dst = edge_index[1]
    ws = W_attn[:, :d_out]
    we = W_attn[:, d_out:d_out + d_edge]
    wd = W_attn[:, d_out + d_edge:]

    z, a_src2, a_dst2 = _tc_dense(h, W_fc, ws, wd)
    ea2 = _tc_ea(edge_attr, we)
    a_src = a_src2.reshape(n)
    a_dst = a_dst2.reshape(n)
    ea = ea2.reshape(e)

    x, dpart = _sc_edge_logits(src, dst, ea, a_src, a_dst)

    ept = e // NW
    nch = ept // K
    xb = lax.bitcast_convert_type(x, jnp.int32)
    ep = jnp.concatenate(
        [src.reshape(NW * nch, 1, K),
         dst.reshape(NW * nch, 1, K),
         xb.reshape(NW * nch, 1, K)], axis=1).reshape(-1)

    hpart = _sc_scatter(ep, dpart, z)
    return _tc_sum(hpart)


# speedup vs baseline: 13.9563x; 1.2990x over previous
import functools

import jax
import jax.numpy as jnp
from jax import lax
from jax.experimental import pallas as pl
from jax.experimental.pallas import tpu as pltpu
from jax.experimental.pallas import tpu_sc as plsc

NC, NS, LANES = 2, 16, 16
NW = NC * NS
NPAD = 10240
SLICE = NPAD // NS
K = 80


def _tc_dense(h, w_fc, ws_row, wd_row):
    n, d = h.shape
    br = 2000

    def body(h_ref, w_ref, ws_ref, wd_ref, z_ref, as_ref, ad_ref):
        zb = lax.dot_general(h_ref[...], w_ref[...],
                             dimension_numbers=(((1,), (1,)), ((), ())),
                             preferred_element_type=jnp.float32)
        zb = jnp.where(zb >= 0.0, zb, 0.2 * zb)
        z_ref[...] = zb
        as_ref[...] = jnp.sum(zb * ws_ref[...], axis=1, keepdims=True)
        ad_ref[...] = jnp.sum(zb * wd_ref[...], axis=1, keepdims=True)

    return pl.pallas_call(
        body,
        grid=(n // br,),
        in_specs=[
            pl.BlockSpec((br, d), lambda i: (i, 0)),
            pl.BlockSpec((d, d), lambda i: (0, 0)),
            pl.BlockSpec((1, d), lambda i: (0, 0)),
            pl.BlockSpec((1, d), lambda i: (0, 0)),
        ],
        out_specs=[
            pl.BlockSpec((br, d), lambda i: (i, 0)),
            pl.BlockSpec((br, 1), lambda i: (i, 0)),
            pl.BlockSpec((br, 1), lambda i: (i, 0)),
        ],
        out_shape=[
            jax.ShapeDtypeStruct((n, d), jnp.float32),
            jax.ShapeDtypeStruct((n, 1), jnp.float32),
            jax.ShapeDtypeStruct((n, 1), jnp.float32),
        ],
    )(h, w_fc, ws_row, wd_row)


def _tc_ea(edge_attr, we_row):
    e, de = edge_attr.shape
    be = 8000

    def body(a_ref, w_ref, o_ref):
        o_ref[...] = jnp.sum(a_ref[...] * w_ref[...], axis=1, keepdims=True)

    return pl.pallas_call(
        body,
        grid=(e // be,),
        in_specs=[
            pl.BlockSpec((be, de), lambda i: (i, 0)),
            pl.BlockSpec((1, de), lambda i: (0, 0)),
        ],
        out_specs=pl.BlockSpec((be, 1), lambda i: (i, 0)),
        out_shape=jax.ShapeDtypeStruct((e, 1), jnp.float32),
    )(edge_attr, we_row)


def _sc_edge_logits(ei, ea, a_src, a_dst):
    e = ei.shape[0] // 2
    n = a_src.shape[0]
    ept = e // NW
    nv = ept // LANES
    gpc = K // LANES
    mesh = plsc.VectorSubcoreMesh(core_axis_name="c", subcore_axis_name="s",
                                  num_cores=NC, num_subcores=NS)

    @functools.partial(
        pl.kernel,
        out_type=[
            jax.ShapeDtypeStruct((3 * e,), jnp.int32),
            jax.ShapeDtypeStruct((NC, NPAD), jnp.float32),
        ],
        mesh=mesh,
        scratch_types=[
            pltpu.VMEM((NPAD,), jnp.float32),
            pltpu.VMEM((NPAD,), jnp.float32),
            pltpu.VMEM((NPAD,), jnp.float32),
            pltpu.VMEM((ept,), jnp.int32),
            pltpu.VMEM((ept,), jnp.int32),
            pltpu.VMEM((ept,), jnp.float32),
            pltpu.VMEM((3 * ept,), jnp.int32),
            pltpu.VMEM((SLICE,), jnp.float32),
            pltpu.VMEM((SLICE,), jnp.float32),
            pltpu.VMEM_SHARED((NS, NPAD), jnp.float32),
        ],
        compiler_params=pltpu.CompilerParams(needs_layout_passes=False),
    )
    def k(ei_h, ea_h, as_h, ad_h, ep_h, dpart_h,
          asrc_v, adst_v, den_v, src_v, dst_v, ea_v, pk_v, red_v, tmp_v, shd):
        cid = lax.axis_index("c")
        sid = lax.axis_index("s")
        base = (cid * NS + sid) * ept
        pltpu.sync_copy(ei_h.at[pl.ds(base, ept)], src_v)
        pltpu.sync_copy(ei_h.at[pl.ds(e + base, ept)], dst_v)
        pltpu.sync_copy(ea_h.at[pl.ds(base, ept)], ea_v)
        pltpu.sync_copy(as_h, asrc_v.at[pl.ds(0, n)])
        pltpu.sync_copy(ad_h, adst_v.at[pl.ds(0, n)])

        zeros = jnp.zeros((LANES,), jnp.float32)

        def zero_step(i, _):
            den_v[pl.ds(i * LANES, LANES)] = zeros
            return 0
        lax.fori_loop(0, NPAD // LANES, zero_step, 0)

        def step(i, _):
            si = src_v[pl.ds(i * LANES, LANES)]
            di = dst_v[pl.ds(i * LANES, LANES)]
            t = (plsc.load_gather(asrc_v, [si]) +
                 plsc.load_gather(adst_v, [di]) +
                 ea_v[pl.ds(i * LANES, LANES)])
            t = jnp.where(t >= 0.0, t, 0.2 * t)
            xv = jnp.exp(t)
            po = (i // gpc) * 3 * K + (i % gpc) * LANES
            pk_v[pl.ds(po, LANES)] = si
            pk_v[pl.ds(po + K, LANES)] = di
            pk_v[pl.ds(po + 2 * K, LANES)] = plsc.bitcast(xv, jnp.int32)
            plsc.addupdate_scatter(den_v, [di], xv)
            return 0
        lax.fori_loop(0, nv, step, 0)

        pltpu.sync_copy(pk_v, ep_h.at[pl.ds(3 * base, 3 * ept)])
        pltpu.sync_copy(den_v, shd.at[sid])
        plsc.subcore_barrier()

        off = sid * SLICE
        pltpu.sync_copy(shd.at[0, pl.ds(off, SLICE)], red_v)
        for j in range(1, NS):
            pltpu.sync_copy(shd.at[j, pl.ds(off, SLICE)], tmp_v)

            def add_step(i, _):
                sl = pl.ds(i * LANES, LANES)
                red_v[sl] = red_v[sl] + tmp_v[sl]
                return 0
            lax.fori_loop(0, SLICE // LANES, add_step, 0)
        pltpu.sync_copy(red_v, dpart_h.at[cid, pl.ds(off, SLICE)])

    return k(ei, ea, a_src, a_dst)


def _sc_scatter(ep, dpart, z):
    n, d = z.shape
    e = ep.shape[0] // 3
    ept = e // NW
    nch = ept // K
    zrows = 16
    mesh = plsc.VectorSubcoreMesh(core_axis_name="c", subcore_axis_name="s",
                                  num_cores=NC, num_subcores=NS)

    @functools.partial(
        pl.kernel,
        out_type=jax.ShapeDtypeStruct((NC, n, d), jnp.float32),
        mesh=mesh,
        scratch_types=[
            pltpu.VMEM((NPAD,), jnp.float32),
            pltpu.VMEM((NPAD,), jnp.float32),
            pltpu.VMEM((3 * K,), jnp.int32),
            pltpu.VMEM((K,), jnp.int32),
            pltpu.VMEM((K,), jnp.int32),
            pltpu.VMEM((K,), jnp.float32),
            pltpu.VMEM((K, 128), jnp.float32),
            pltpu.VMEM((3 * K,), jnp.int32),
            pltpu.VMEM((K,), jnp.int32),
            pltpu.VMEM((K,), jnp.int32),
            pltpu.VMEM((K,), jnp.float32),
            pltpu.VMEM((K, 128), jnp.float32),
            pltpu.VMEM((zrows, 128), jnp.float32),
            pltpu.SemaphoreType.DMA,
            pltpu.SemaphoreType.DMA,
            pltpu.VMEM_SHARED((n, 128), jnp.float32),
        ],
        compiler_params=pltpu.CompilerParams(needs_layout_passes=False),
    )
    def k(ep_h, dp_h, z_h, hp_h,
          den_v, tmp_v, ebufa, srca, dsta, ala, rowsa,
          ebufb, srcb, dstb, alb, rowsb, zb, sema, semb, shacc):
        cid = lax.axis_index("c")
        sid = lax.axis_index("s")
        wid = cid * NS + sid

        pltpu.sync_copy(dp_h.at[0], den_v)
        pltpu.sync_copy(dp_h.at[1], tmp_v)

        def den_step(i, _):
            sl = pl.ds(i * LANES, LANES)
            dv = den_v[sl] + tmp_v[sl]
            den_v[sl] = jnp.where(dv == 0.0, 1.0, dv)
            return 0
        lax.fori_loop(0, NPAD // LANES, den_step, 0)

        zeros = jnp.zeros((LANES,), jnp.float32)

        def zb_step(i, _):
            zb[i // 8, pl.ds((i % 8) * LANES, LANES)] = zeros
            return 0
        lax.fori_loop(0, zrows * 8, zb_step, 0)
        r0 = sid * 624
        nq = jnp.where(sid == NS - 1, 40, 39)

        def zero_out(q, _):
            pltpu.sync_copy(zb, shacc.at[pl.ds(r0 + q * zrows, zrows)])
            return 0
        lax.fori_loop(0, nq, zero_out, 0)
        plsc.subcore_barrier()

        def load_scalars(c, ebuf, srcc, dstc, al):
            pltpu.sync_copy(ep_h.at[pl.ds((wid * nch + c) * 3 * K, 3 * K)], ebuf)

            def cp_step(g, _):
                sl = pl.ds(g * LANES, LANES)
                srcc[sl] = ebuf[pl.ds(g * LANES, LANES)]
                di = ebuf[pl.ds(K + g * LANES, LANES)]
                dstc[sl] = di
                xv = plsc.bitcast(ebuf[pl.ds(2 * K + g * LANES, LANES)],
                                  jnp.float32)
                al[sl] = xv / plsc.load_gather(den_v, [di])
                return 0
            lax.fori_loop(0, K // LANES, cp_step, 0)

        def scale_scatter(dstc, al, rows):
            def row_step(r, _):
                av = plsc.load_gather(al, [jnp.full((LANES,), r, jnp.int32)])
                for f in range(8):
                    sl = pl.ds(f * LANES, LANES)
                    rows[r, sl] = rows[r, sl] * av
                return 0
            lax.fori_loop(0, K, row_step, 0)
            pltpu.sync_copy(rows, shacc.at[dstc], add=True)

        load_scalars(0, ebufa, srca, dsta, ala)
        pltpu.async_copy(z_h.at[srca], rowsa, sema)

        def pair(j, _):
            load_scalars(2 * j + 1, ebufb, srcb, dstb, alb)
            pltpu.make_async_copy(z_h.at[srca], rowsa, sema).wait()
            pltpu.async_copy(z_h.at[srcb], rowsb, semb)
            scale_scatter(dsta, ala, rowsa)
            load_scalars(2 * j + 2, ebufa, srca, dsta, ala)
            pltpu.make_async_copy(z_h.at[srcb], rowsb, semb).wait()
            pltpu.async_copy(z_h.at[srca], rowsa, sema)
            scale_scatter(dstb, alb, rowsb)
            return 0
        lax.fori_loop(0, (nch - 1) // 2, pair, 0)
        pltpu.make_async_copy(z_h.at[srca], rowsa, sema).wait()
        scale_scatter(dsta, ala, rowsa)
        plsc.subcore_barrier()

        def write_out(q, _):
            pltpu.sync_copy(shacc.at[pl.ds(r0 + q * zrows, zrows)], zb)
            pltpu.sync_copy(zb, hp_h.at[cid, pl.ds(r0 + q * zrows, zrows)])
            return 0
        lax.fori_loop(0, nq, write_out, 0)

    return k(ep, dpart, z)


def _tc_sum(hpart):
    _, n, d = hpart.shape
    br = 2000

    def body(a_ref, b_ref, o_ref):
        o_ref[...] = a_ref[0] + b_ref[0]

    return pl.pallas_call(
        body,
        grid=(n // br,),
        in_specs=[
            pl.BlockSpec((1, br, d), lambda i: (0, i, 0)),
            pl.BlockSpec((1, br, d), lambda i: (1, i, 0)),
        ],
        out_specs=pl.BlockSpec((br, d), lambda i: (i, 0)),
        out_shape=jax.ShapeDtypeStruct((n, d), jnp.float32),
    )(hpart, hpart)


def kernel(h, edge_index, edge_attr, W_fc, W_attn):
    n, d_in = h.shape
    e = edge_index.shape[1]
    d_out = W_fc.shape[0]
    d_edge = edge_attr.shape[1]
    ws = W_attn[:, :d_out]
    we = W_attn[:, d_out:d_out + d_edge]
    wd = W_attn[:, d_out + d_edge:]

    z, a_src2, a_dst2 = _tc_dense(h, W_fc, ws, wd)
    ea2 = _tc_ea(edge_attr, we)
    a_src = a_src2.reshape(n)
    a_dst = a_dst2.reshape(n)
    ea = ea2.reshape(e)

    ep, dpart = _sc_edge_logits(edge_index.reshape(2 * e), ea, a_src, a_dst)
    hpart = _sc_scatter(ep, dpart, z)
    return _tc_sum(hpart)


# speedup vs baseline: 17.3107x; 1.2403x over previous
import functools

import jax
import jax.numpy as jnp
from jax import lax
from jax.experimental import pallas as pl
from jax.experimental.pallas import tpu as pltpu
from jax.experimental.pallas import tpu_sc as plsc

NC, NS, LANES = 2, 16, 16
NW = NC * NS
NPAD = 10240
SLICE = NPAD // NS
K = 80


def _tc_dense(h, w_fc, ws_row, wd_row):
    n, d = h.shape
    br = 2000

    def body(h_ref, w_ref, ws_ref, wd_ref, z_ref, as_ref, ad_ref):
        zb = lax.dot_general(h_ref[...], w_ref[...],
                             dimension_numbers=(((1,), (1,)), ((), ())),
                             preferred_element_type=jnp.float32)
        zb = jnp.where(zb >= 0.0, zb, 0.2 * zb)
        z_ref[...] = zb
        as_ref[...] = jnp.sum(zb * ws_ref[...], axis=1, keepdims=True)
        ad_ref[...] = jnp.sum(zb * wd_ref[...], axis=1, keepdims=True)

    return pl.pallas_call(
        body,
        grid=(n // br,),
        in_specs=[
            pl.BlockSpec((br, d), lambda i: (i, 0)),
            pl.BlockSpec((d, d), lambda i: (0, 0)),
            pl.BlockSpec((1, d), lambda i: (0, 0)),
            pl.BlockSpec((1, d), lambda i: (0, 0)),
        ],
        out_specs=[
            pl.BlockSpec((br, d), lambda i: (i, 0)),
            pl.BlockSpec((br, 1), lambda i: (i, 0)),
            pl.BlockSpec((br, 1), lambda i: (i, 0)),
        ],
        out_shape=[
            jax.ShapeDtypeStruct((n, d), jnp.float32),
            jax.ShapeDtypeStruct((n, 1), jnp.float32),
            jax.ShapeDtypeStruct((n, 1), jnp.float32),
        ],
    )(h, w_fc, ws_row, wd_row)


def _tc_ea(edge_attr, we_row):
    e, de = edge_attr.shape
    be = 8000

    def body(a_ref, w_ref, o_ref):
        o_ref[...] = jnp.sum(a_ref[...] * w_ref[...], axis=1, keepdims=True)

    return pl.pallas_call(
        body,
        grid=(e // be,),
        in_specs=[
            pl.BlockSpec((be, de), lambda i: (i, 0)),
            pl.BlockSpec((1, de), lambda i: (0, 0)),
        ],
        out_specs=pl.BlockSpec((be, 1), lambda i: (i, 0)),
        out_shape=jax.ShapeDtypeStruct((e, 1), jnp.float32),
    )(edge_attr, we_row)


def _sc_edge_logits(ei, ea, a_src, a_dst):
    e = ei.shape[0] // 2
    n = a_src.shape[0]
    ept = e // NW
    nv = ept // LANES
    gpc = K // LANES
    mesh = plsc.VectorSubcoreMesh(core_axis_name="c", subcore_axis_name="s",
                                  num_cores=NC, num_subcores=NS)

    @functools.partial(
        pl.kernel,
        out_type=[
            jax.ShapeDtypeStruct((3 * e + 9 * K,), jnp.int32),
            jax.ShapeDtypeStruct((NC, NPAD), jnp.float32),
        ],
        mesh=mesh,
        scratch_types=[
            pltpu.VMEM((NPAD,), jnp.float32),
            pltpu.VMEM((NPAD,), jnp.float32),
            pltpu.VMEM((NPAD,), jnp.float32),
            pltpu.VMEM((ept,), jnp.int32),
            pltpu.VMEM((ept,), jnp.int32),
            pltpu.VMEM((ept,), jnp.float32),
            pltpu.VMEM((3 * ept,), jnp.int32),
            pltpu.VMEM((SLICE,), jnp.float32),
            pltpu.VMEM((SLICE,), jnp.float32),
            pltpu.SemaphoreType.DMA,
            pltpu.VMEM_SHARED((NS, NPAD), jnp.float32),
        ],
        compiler_params=pltpu.CompilerParams(needs_layout_passes=False),
    )
    def k(ei_h, ea_h, as_h, ad_h, ep_h, dpart_h,
          asrc_v, adst_v, den_v, src_v, dst_v, ea_v, pk_v, red_v, tmp_v, sem,
          shd):
        cid = lax.axis_index("c")
        sid = lax.axis_index("s")
        base = (cid * NS + sid) * ept
        inputs = [
            (ei_h.at[pl.ds(base, ept)], src_v),
            (ei_h.at[pl.ds(e + base, ept)], dst_v),
            (ea_h.at[pl.ds(base, ept)], ea_v),
            (as_h, asrc_v.at[pl.ds(0, n)]),
            (ad_h, adst_v.at[pl.ds(0, n)]),
        ]
        for s, t in inputs:
            pltpu.async_copy(s, t, sem)

        zeros = jnp.zeros((LANES,), jnp.float32)

        def zero_step(i, _):
            den_v[pl.ds(i * LANES, LANES)] = zeros
            return 0
        lax.fori_loop(0, NPAD // LANES, zero_step, 0)
        for s, t in inputs:
            pltpu.make_async_copy(s, t, sem).wait()

        def step(i, _):
            si = src_v[pl.ds(i * LANES, LANES)]
            di = dst_v[pl.ds(i * LANES, LANES)]
            t = (plsc.load_gather(asrc_v, [si]) +
                 plsc.load_gather(adst_v, [di]) +
                 ea_v[pl.ds(i * LANES, LANES)])
            t = jnp.where(t >= 0.0, t, 0.2 * t)
            xv = jnp.exp(t)
            po = (i // gpc) * 3 * K + (i % gpc) * LANES
            pk_v[pl.ds(po, LANES)] = si
            pk_v[pl.ds(po + K, LANES)] = di
            pk_v[pl.ds(po + 2 * K, LANES)] = plsc.bitcast(xv, jnp.int32)
            plsc.addupdate_scatter(den_v, [di], xv)
            return 0
        lax.fori_loop(0, nv, step, 0)

        pltpu.sync_copy(pk_v, ep_h.at[pl.ds(3 * base, 3 * ept)])
        pltpu.sync_copy(den_v, shd.at[sid])
        plsc.subcore_barrier()

        off = sid * SLICE
        pltpu.sync_copy(shd.at[0, pl.ds(off, SLICE)], red_v)
        for j in range(1, NS):
            pltpu.sync_copy(shd.at[j, pl.ds(off, SLICE)], tmp_v)

            def add_step(i, _):
                sl = pl.ds(i * LANES, LANES)
                red_v[sl] = red_v[sl] + tmp_v[sl]
                return 0
            lax.fori_loop(0, SLICE // LANES, add_step, 0)
        pltpu.sync_copy(red_v, dpart_h.at[cid, pl.ds(off, SLICE)])

    return k(ei, ea, a_src, a_dst)


def _sc_scatter(ep, dpart, z, e):
    n, d = z.shape
    ept = e // NW
    nch = ept // K
    zrows = 16
    mesh = plsc.VectorSubcoreMesh(core_axis_name="c", subcore_axis_name="s",
                                  num_cores=NC, num_subcores=NS)

    @functools.partial(
        pl.kernel,
        out_type=jax.ShapeDtypeStruct((NC, n, d), jnp.float32),
        mesh=mesh,
        scratch_types=[
            pltpu.VMEM((NPAD,), jnp.float32),
            pltpu.VMEM((2048,), jnp.float32),
            pltpu.VMEM((zrows, 128), jnp.float32),
        ] + 3 * [
            pltpu.VMEM((3 * K,), jnp.int32),
            pltpu.VMEM((K,), jnp.int32),
            pltpu.VMEM((K,), jnp.int32),
            pltpu.VMEM((K,), jnp.float32),
            pltpu.VMEM((K, 128), jnp.float32),
            pltpu.SemaphoreType.DMA,
            pltpu.SemaphoreType.DMA,
            pltpu.SemaphoreType.DMA,
        ] + [
            pltpu.VMEM_SHARED((n, 128), jnp.float32),
        ],
        compiler_params=pltpu.CompilerParams(needs_layout_passes=False),
    )
    def k(ep_h, dp_h, z_h, hp_h, den_v, stg, zb, *rest):
        bufs = (rest[0:8], rest[8:16], rest[16:24])
        shacc = rest[24]
        cid = lax.axis_index("c")
        sid = lax.axis_index("s")
        wid = cid * NS + sid

        pltpu.sync_copy(dp_h.at[0], den_v)
        for p in range(NPAD // 2048):
            pltpu.sync_copy(dp_h.at[1, pl.ds(p * 2048, 2048)], stg)

            def den_step(i, _, p=p):
                sl = pl.ds(p * 2048 + i * LANES, LANES)
                dv = den_v[sl] + stg[pl.ds(i * LANES, LANES)]
                den_v[sl] = jnp.where(dv == 0.0, 1.0, dv)
                return 0
            lax.fori_loop(0, 2048 // LANES, den_step, 0)

        zeros = jnp.zeros((LANES,), jnp.float32)

        def zb_step(i, _):
            zb[i // 8, pl.ds((i % 8) * LANES, LANES)] = zeros
            return 0
        lax.fori_loop(0, zrows * 8, zb_step, 0)
        r0 = sid * 624
        nq = jnp.where(sid == NS - 1, 40, 39)

        def zero_out(q, _):
            pltpu.sync_copy(zb, shacc.at[pl.ds(r0 + q * zrows, zrows)])
            return 0
        lax.fori_loop(0, nq, zero_out, 0)
        plsc.subcore_barrier()

        def ep_slice(c):
            return ep_h.at[pl.ds((wid * nch + c) * 3 * K, 3 * K)]

        def start_ebuf(c, bf):
            pltpu.async_copy(ep_slice(c), bf[0], bf[7])

        def wait_ebuf(c, bf):
            pltpu.make_async_copy(ep_slice(c), bf[0], bf[7]).wait()

        def load_scalars(c, bf):
            ebuf, srcc, dstc, al = bf[0], bf[1], bf[2], bf[3]
            wait_ebuf(c, bf)
            for g in range(K // LANES):
                sl = pl.ds(g * LANES, LANES)
                srcc[sl] = ebuf[sl]
                di = ebuf[pl.ds(K + g * LANES, LANES)]
                dstc[sl] = di
                xv = plsc.bitcast(ebuf[pl.ds(2 * K + g * LANES, LANES)],
                                  jnp.float32)
                al[sl] = xv / plsc.load_gather(den_v, [di])
            start_ebuf(c + 3, bf)

        def start_gather(bf):
            pltpu.async_copy(z_h.at[bf[1]], bf[4], bf[5])

        def wait_gather(bf):
            pltpu.make_async_copy(z_h.at[bf[1]], bf[4], bf[5]).wait()

        def scale(bf):
            al, rows = bf[3], bf[4]

            @plsc.parallel_loop(0, K // 4, unroll=2)
            def row4(r):
                rr = r * 4
                avs = [plsc.load_gather(al,
                                        [jnp.full((LANES,), rr + u, jnp.int32)])
                       for u in range(4)]
                for f in range(8):
                    sl = pl.ds(f * LANES, LANES)
                    for u in range(4):
                        rows[rr + u, sl] = rows[rr + u, sl] * avs[u]

        def start_scatter(bf):
            pltpu.async_copy(bf[4], shacc.at[bf[2]], bf[6], add=True)

        def wait_scatter(bf):
            pltpu.make_async_copy(bf[4], shacc.at[bf[2]], bf[6]).wait()

        b0, b1, b2 = bufs
        start_ebuf(0, b0)
        start_ebuf(1, b1)
        start_ebuf(2, b2)
        load_scalars(0, b0)
        start_gather(b0)
        load_scalars(1, b1)
        start_gather(b1)
        wait_gather(b0)
        scale(b0)
        start_scatter(b0)
        load_scalars(2, b2)
        start_gather(b2)
        wait_gather(b1)
        scale(b1)
        start_scatter(b1)
        wait_scatter(b0)
        load_scalars(3, b0)
        start_gather(b0)

        def group(gi, _):
            i = 3 * gi + 2
            wait_gather(b2)
            scale(b2)
            start_scatter(b2)
            wait_scatter(b1)
            load_scalars(i + 2, b1)
            start_gather(b1)
            wait_gather(b0)
            scale(b0)
            start_scatter(b0)
            wait_scatter(b2)
            load_scalars(i + 3, b2)
            start_gather(b2)
            wait_gather(b1)
            scale(b1)
            start_scatter(b1)
            wait_scatter(b0)
            load_scalars(i + 4, b0)
            start_gather(b0)
            return 0
        lax.fori_loop(0, (nch - 5) // 3, group, 0)
        wait_gather(b2)
        scale(b2)
        start_scatter(b2)
        wait_scatter(b1)
        load_scalars(nch - 1, b1)
        start_gather(b1)
        wait_gather(b0)
        scale(b0)
        start_scatter(b0)
        wait_gather(b1)
        scale(b1)
        start_scatter(b1)
        wait_scatter(b2)
        wait_scatter(b0)
        wait_scatter(b1)
        wait_ebuf(nch, b2)
        wait_ebuf(nch + 1, b0)
        wait_ebuf(nch + 2, b1)
        plsc.subcore_barrier()

        def write_out(q, _):
            pltpu.sync_copy(shacc.at[pl.ds(r0 + q * zrows, zrows)], zb)
            pltpu.sync_copy(zb, hp_h.at[cid, pl.ds(r0 + q * zrows, zrows)])
            return 0
        lax.fori_loop(0, nq, write_out, 0)

    return k(ep, dpart, z)


def _tc_sum(hpart):
    _, n, d = hpart.shape
    br = 2000

    def body(a_ref, b_ref, o_ref):
        o_ref[...] = a_ref[0] + b_ref[0]

    return pl.pallas_call(
        body,
        grid=(n // br,),
        in_specs=[
            pl.BlockSpec((1, br, d), lambda i: (0, i, 0)),
            pl.BlockSpec((1, br, d), lambda i: (1, i, 0)),
        ],
        out_specs=pl.BlockSpec((br, d), lambda i: (i, 0)),
        out_shape=jax.ShapeDtypeStruct((n, d), jnp.float32),
    )(hpart, hpart)


def kernel(h, edge_index, edge_attr, W_fc, W_attn):
    n, d_in = h.shape
    e = edge_index.shape[1]
    d_out = W_fc.shape[0]
    d_edge = edge_attr.shape[1]
    ws = W_attn[:, :d_out]
    we = W_attn[:, d_out:d_out + d_edge]
    wd = W_attn[:, d_out + d_edge:]

    z, a_src2, a_dst2 = _tc_dense(h, W_fc, ws, wd)
    ea2 = _tc_ea(edge_attr, we)
    a_src = a_src2.reshape(n)
    a_dst = a_dst2.reshape(n)
    ea = ea2.reshape(e)

    ep, dpart = _sc_edge_logits(edge_index.reshape(2 * e), ea, a_src, a_dst)
    hpart = _sc_scatter(ep, dpart, z, e)
    return _tc_sum(hpart)
